# baseline (speedup 1.0000x reference)
import functools
import jax
import jax.numpy as jnp
from jax import lax
from jax.experimental import pallas as pl
from jax.experimental.pallas import tpu as pltpu
from jax.experimental.pallas import tpu_sc as plsc

N = 10000
E = 320000
DIN = 128
DH = 64
G = 64

NC = 2
NS = 16
NW = NC * NS
EPW = E // NW
CH = 80
NCHUNK = EPW // CH
NBUF = 5
NSET = 2
NGROUP = NCHUNK // NBUF
NIT = 10
NPT = N // NIT

def _sc_deg_body(dst_hbm, zero_hbm, out_hbm, dst_t, ones_v, deg_sh, sem):
    c = lax.axis_index("c")
    s = lax.axis_index("s")
    wid = c * NS + s
    cp = pltpu.async_copy(dst_hbm.at[wid], dst_t, sem)
    one = jnp.ones((16,), jnp.float32)
    for i in range(CH // 16):
        ones_v[pl.ds(i * 16, 16)] = one

    @pl.when(s == 0)
    def _():
        pltpu.sync_copy(zero_hbm, deg_sh)

    cp.wait()
    plsc.subcore_barrier()

    def body(k, carry):
        pltpu.async_copy(ones_v, deg_sh.at[dst_t.at[k]], sem, add=True)
        return carry

    lax.fori_loop(0, NCHUNK, body, 0)

    def drain(k, carry):
        pltpu.make_async_copy(ones_v, deg_sh.at[pl.ds(0, CH)], sem).wait()
        return carry

    lax.fori_loop(0, NCHUNK, drain, 0)
    plsc.subcore_barrier()

    @pl.when(s == 0)
    def _():
        pltpu.sync_copy(deg_sh, out_hbm.at[c])


def _sc_scatter_body(xs_hbm, src_hbm, dst_hbm, zero_hbm, out_hbm,
                     src_t, dst_t, rows_v, acc_sh, sem, gsem, ssem):
    c = lax.axis_index("c")
    s = lax.axis_index("s")
    wid = c * NS + s
    cp_s = pltpu.async_copy(src_hbm.at[wid], src_t, sem)
    cp_d = pltpu.async_copy(dst_hbm.at[wid], dst_t, sem)

    @pl.when(s < NIT)
    def _():
        pltpu.sync_copy(zero_hbm.at[pl.ds(s * NPT, NPT)],
                        acc_sh.at[pl.ds(s * NPT, NPT)])

    def fire(g, setoff):
        for b in range(NBUF):
            pltpu.async_copy(xs_hbm.at[src_t.at[g * NBUF + b]],
                             rows_v.at[setoff + b], gsem)

    cp_s.wait()
    for gg in range(NSET - 1):
        fire(gg, gg * NBUF)
    cp_d.wait()
    plsc.subcore_barrier()

    def drain_scat(setoff):
        for b in range(NBUF):
            pltpu.make_async_copy(rows_v.at[setoff + b],
                                  acc_sh.at[pl.ds(0, CH)], ssem).wait()

    def group(g, carry):
        setoff = (g % NSET) * NBUF
        other = ((g + 1) % NSET) * NBUF

        @pl.when(g >= 1)
        def _():
            drain_scat(other)

        @pl.when(g + NSET - 1 < NGROUP)
        def _():
            fire(g + NSET - 1, ((g + NSET - 1) % NSET) * NBUF)

        for b in range(NBUF):
            pltpu.make_async_copy(xs_hbm.at[src_t.at[0]],
                                  rows_v.at[setoff + b], gsem).wait()
            pltpu.async_copy(rows_v.at[setoff + b],
                             acc_sh.at[dst_t.at[g * NBUF + b]], ssem,
                             add=True)

        return carry

    lax.fori_loop(0, NGROUP, group, 0)
    drain_scat(((NGROUP - 1) % NSET) * NBUF)
    plsc.subcore_barrier()

    @pl.when(s < NIT)
    def _():
        pltpu.sync_copy(acc_sh.at[pl.ds(s * NPT, NPT)],
                        out_hbm.at[c, pl.ds(s * NPT, NPT)])


@functools.cache
def _sc_kernels():
    mesh = plsc.VectorSubcoreMesh(core_axis_name="c", subcore_axis_name="s",
                                  num_cores=NC, num_subcores=NS)
    sc_deg = pl.kernel(
        _sc_deg_body,
        out_type=jax.ShapeDtypeStruct((NC, N), jnp.float32),
        mesh=mesh,
        compiler_params=pltpu.CompilerParams(use_tc_tiling_on_sc=False),
        scratch_types=[
            pltpu.VMEM((NCHUNK, CH), jnp.int32),
            pltpu.VMEM((CH,), jnp.float32),
            pltpu.VMEM_SHARED((N,), jnp.float32),
            pltpu.SemaphoreType.DMA,
        ],
    )
    sc_scatter = pl.kernel(
        _sc_scatter_body,
        out_type=jax.ShapeDtypeStruct((NC, N, DH), jnp.float32),
        mesh=mesh,
        compiler_params=pltpu.CompilerParams(use_tc_tiling_on_sc=False),
        scratch_types=[
            pltpu.VMEM((NCHUNK, CH), jnp.int32),
            pltpu.VMEM((NCHUNK, CH), jnp.int32),
            pltpu.VMEM((NSET * NBUF, CH, DH), jnp.float32),
            pltpu.VMEM_SHARED((N, DH), jnp.float32),
            pltpu.SemaphoreType.DMA,
            pltpu.SemaphoreType.DMA,
            pltpu.SemaphoreType.DMA,
        ],
    )
    return sc_deg, sc_scatter


def _run_sc_deg(dst3, zero_n):
    return _sc_kernels()[0](dst3, zero_n)


def _run_sc_scatter(xs, src3, dst3, zero_nd):
    return _sc_kernels()[1](xs, src3, dst3, zero_nd)


def _tc1_body(x_ref, w1_ref, degp_ref, dis_ref, xs_ref):
    deg = degp_ref[0] + degp_ref[1] + 1.0
    dis = lax.rsqrt(deg)
    xw = jnp.dot(x_ref[...], w1_ref[...], preferred_element_type=jnp.float32)
    dis_ref[...] = dis
    xs_ref[...] = xw * dis


_tc1 = pl.pallas_call(
    _tc1_body,
    out_shape=(jax.ShapeDtypeStruct((N, 1), jnp.float32),
               jax.ShapeDtypeStruct((N, DH), jnp.float32)),
)


def _tc2_body(accp_ref, xs1_ref, dis_ref, b1_ref, w2_ref, xs2_ref):
    acc = accp_ref[0] + accp_ref[1] + xs1_ref[...]
    h1 = jnp.maximum(acc * dis_ref[...] + b1_ref[...], 0.0)
    xw2 = jnp.dot(h1, w2_ref[...], preferred_element_type=jnp.float32)
    xs2_ref[...] = xw2 * dis_ref[...]


_tc2 = pl.pallas_call(
    _tc2_body,
    out_shape=jax.ShapeDtypeStruct((N, DH), jnp.float32),
)


def _tc3_body(accp_ref, xs2_ref, dis_ref, b2_ref, batch_ref, w3_ref, b3_ref,
              out_ref):
    acc = accp_ref[0] + accp_ref[1] + xs2_ref[...]
    h2 = jnp.maximum(acc * dis_ref[...] + b2_ref[...], 0.0)
    oh = (batch_ref[...] == lax.broadcasted_iota(jnp.int32, (N, G), 1)
          ).astype(jnp.float32)
    sums = lax.dot_general(oh, h2, (((0,), (0,)), ((), ())),
                           preferred_element_type=jnp.float32)
    counts = lax.dot_general(oh, jnp.ones((N, 1), jnp.float32),
                             (((0,), (0,)), ((), ())),
                             preferred_element_type=jnp.float32)
    pooled = sums / jnp.maximum(counts, 1.0)
    out_ref[...] = jnp.dot(pooled, w3_ref[...],
                           preferred_element_type=jnp.float32) + b3_ref[...]


_tc3 = pl.pallas_call(
    _tc3_body,
    out_shape=jax.ShapeDtypeStruct((G, 2), jnp.float32),
)


def kernel(x, edge_index, batch, W1, b1, W2, b2, W3, b3):
    src = edge_index[0].reshape(NW, NCHUNK, CH)
    dst = edge_index[1].reshape(NW, NCHUNK, CH)
    zero_n = jnp.zeros((N,), jnp.float32)
    zero_nd = jnp.zeros((N, DH), jnp.float32)
    degp = _run_sc_deg(dst, zero_n)
    dis, xs1 = _tc1(x, W1, degp.reshape(NC, N, 1))
    accp1 = _run_sc_scatter(xs1, src, dst, zero_nd)
    xs2 = _tc2(accp1, xs1, dis, b1.reshape(1, DH), W2)
    accp2 = _run_sc_scatter(xs2, src, dst, zero_nd)
    return _tc3(accp2, xs2, dis, b2.reshape(1, DH), batch.reshape(N, 1),
                W3, b3.reshape(1, 2))


# speedup vs baseline: 1.0089x; 1.0089x over previous
import functools
import jax
import jax.numpy as jnp
from jax import lax
from jax.experimental import pallas as pl
from jax.experimental.pallas import tpu as pltpu
from jax.experimental.pallas import tpu_sc as plsc

N = 10000
E = 320000
DIN = 128
DH = 64
G = 64

NC = 2
NS = 16
NW = NC * NS
EPW = E // NW
CH = 80
NCHUNK = EPW // CH
NBUF = 5
NSET = 2
NGROUP = NCHUNK // NBUF
NIT = 10
NPT = N // NIT

def _sc_deg_body(dst_hbm, zero_hbm, out_hbm, dst_t, ones_v, deg_sh, sem):
    c = lax.axis_index("c")
    s = lax.axis_index("s")
    wid = c * NS + s
    cp = pltpu.async_copy(dst_hbm.at[wid], dst_t, sem)
    one = jnp.ones((16,), jnp.float32)
    for i in range(CH // 16):
        ones_v[pl.ds(i * 16, 16)] = one

    @pl.when(s == 0)
    def _():
        pltpu.sync_copy(zero_hbm, deg_sh)

    cp.wait()
    plsc.subcore_barrier()

    def body(k, carry):
        pltpu.async_copy(ones_v, deg_sh.at[dst_t.at[k]], sem, add=True)
        return carry

    lax.fori_loop(0, NCHUNK, body, 0)

    def drain(k, carry):
        pltpu.make_async_copy(ones_v, deg_sh.at[pl.ds(0, CH)], sem).wait()
        return carry

    lax.fori_loop(0, NCHUNK, drain, 0)
    plsc.subcore_barrier()

    @pl.when(s == 0)
    def _():
        pltpu.sync_copy(deg_sh, out_hbm.at[c])


def _sc_scatter_body(xs_hbm, src_hbm, dst_hbm, zero_hbm, out_hbm,
                     src_t, dst_t, rows_v, acc_sh, sem, gsem, ssem):
    c = lax.axis_index("c")
    s = lax.axis_index("s")
    wid = c * NS + s
    cp_s = pltpu.async_copy(src_hbm.at[wid], src_t, sem)
    cp_d = pltpu.async_copy(dst_hbm.at[wid], dst_t, sem)

    @pl.when(s < NIT)
    def _():
        pltpu.sync_copy(zero_hbm.at[pl.ds(s * NPT, NPT)],
                        acc_sh.at[pl.ds(s * NPT, NPT)])

    def fire(g, setoff):
        for b in range(NBUF):
            pltpu.async_copy(xs_hbm.at[src_t.at[g * NBUF + b]],
                             rows_v.at[setoff + b], gsem)

    cp_s.wait()
    for gg in range(NSET - 1):
        fire(gg, gg * NBUF)
    cp_d.wait()
    plsc.subcore_barrier()

    def group(g, carry):
        setoff = (g % NSET) * NBUF

        @pl.when(g + NSET - 1 < NGROUP)
        def _():
            fire(g + NSET - 1, ((g + NSET - 1) % NSET) * NBUF)

        for b in range(NBUF):
            pltpu.make_async_copy(xs_hbm.at[src_t.at[0]],
                                  rows_v.at[setoff + b], gsem).wait()
            pltpu.sync_copy(rows_v.at[setoff + b],
                            acc_sh.at[dst_t.at[g * NBUF + b]], add=True)

        return carry

    lax.fori_loop(0, NGROUP, group, 0)
    plsc.subcore_barrier()

    @pl.when(s < NIT)
    def _():
        pltpu.sync_copy(acc_sh.at[pl.ds(s * NPT, NPT)],
                        out_hbm.at[c, pl.ds(s * NPT, NPT)])


@functools.cache
def _sc_kernels():
    mesh = plsc.VectorSubcoreMesh(core_axis_name="c", subcore_axis_name="s",
                                  num_cores=NC, num_subcores=NS)
    sc_deg = pl.kernel(
        _sc_deg_body,
        out_type=jax.ShapeDtypeStruct((NC, N), jnp.float32),
        mesh=mesh,
        compiler_params=pltpu.CompilerParams(use_tc_tiling_on_sc=False),
        scratch_types=[
            pltpu.VMEM((NCHUNK, CH), jnp.int32),
            pltpu.VMEM((CH,), jnp.float32),
            pltpu.VMEM_SHARED((N,), jnp.float32),
            pltpu.SemaphoreType.DMA,
        ],
    )
    sc_scatter = pl.kernel(
        _sc_scatter_body,
        out_type=jax.ShapeDtypeStruct((NC, N, DH), jnp.float32),
        mesh=mesh,
        compiler_params=pltpu.CompilerParams(use_tc_tiling_on_sc=False),
        scratch_types=[
            pltpu.VMEM((NCHUNK, CH), jnp.int32),
            pltpu.VMEM((NCHUNK, CH), jnp.int32),
            pltpu.VMEM((NSET * NBUF, CH, DH), jnp.float32),
            pltpu.VMEM_SHARED((N, DH), jnp.float32),
            pltpu.SemaphoreType.DMA,
            pltpu.SemaphoreType.DMA,
            pltpu.SemaphoreType.DMA,
        ],
    )
    return sc_deg, sc_scatter


def _run_sc_deg(dst3, zero_n):
    return _sc_kernels()[0](dst3, zero_n)


def _run_sc_scatter(xs, src3, dst3, zero_nd):
    return _sc_kernels()[1](xs, src3, dst3, zero_nd)


def _tc1_body(x_ref, w1_ref, degp_ref, dis_ref, xs_ref):
    deg = degp_ref[0] + degp_ref[1] + 1.0
    dis = lax.rsqrt(deg)
    xw = jnp.dot(x_ref[...], w1_ref[...], preferred_element_type=jnp.float32)
    dis_ref[...] = dis
    xs_ref[...] = xw * dis


_tc1 = pl.pallas_call(
    _tc1_body,
    out_shape=(jax.ShapeDtypeStruct((N, 1), jnp.float32),
               jax.ShapeDtypeStruct((N, DH), jnp.float32)),
)


def _tc2_body(accp_ref, xs1_ref, dis_ref, b1_ref, w2_ref, xs2_ref):
    acc = accp_ref[0] + accp_ref[1] + xs1_ref[...]
    h1 = jnp.maximum(acc * dis_ref[...] + b1_ref[...], 0.0)
    xw2 = jnp.dot(h1, w2_ref[...], preferred_element_type=jnp.float32)
    xs2_ref[...] = xw2 * dis_ref[...]


_tc2 = pl.pallas_call(
    _tc2_body,
    out_shape=jax.ShapeDtypeStruct((N, DH), jnp.float32),
)


def _tc3_body(accp_ref, xs2_ref, dis_ref, b2_ref, batch_ref, w3_ref, b3_ref,
              out_ref):
    acc = accp_ref[0] + accp_ref[1] + xs2_ref[...]
    h2 = jnp.maximum(acc * dis_ref[...] + b2_ref[...], 0.0)
    oh = (batch_ref[...] == lax.broadcasted_iota(jnp.int32, (N, G), 1)
          ).astype(jnp.float32)
    sums = lax.dot_general(oh, h2, (((0,), (0,)), ((), ())),
                           preferred_element_type=jnp.float32)
    counts = lax.dot_general(oh, jnp.ones((N, 1), jnp.float32),
                             (((0,), (0,)), ((), ())),
                             preferred_element_type=jnp.float32)
    pooled = sums / jnp.maximum(counts, 1.0)
    out_ref[...] = jnp.dot(pooled, w3_ref[...],
                           preferred_element_type=jnp.float32) + b3_ref[...]


_tc3 = pl.pallas_call(
    _tc3_body,
    out_shape=jax.ShapeDtypeStruct((G, 2), jnp.float32),
)


def kernel(x, edge_index, batch, W1, b1, W2, b2, W3, b3):
    src = edge_index[0].reshape(NW, NCHUNK, CH)
    dst = edge_index[1].reshape(NW, NCHUNK, CH)
    zero_n = jnp.zeros((N,), jnp.float32)
    zero_nd = jnp.zeros((N, DH), jnp.float32)
    degp = _run_sc_deg(dst, zero_n)
    dis, xs1 = _tc1(x, W1, degp.reshape(NC, N, 1))
    accp1 = _run_sc_scatter(xs1, src, dst, zero_nd)
    xs2 = _tc2(accp1, xs1, dis, b1.reshape(1, DH), W2)
    accp2 = _run_sc_scatter(xs2, src, dst, zero_nd)
    return _tc3(accp2, xs2, dis, b2.reshape(1, DH), batch.reshape(N, 1),
                W3, b3.reshape(1, 2))
